# Initial kernel scaffold; baseline (speedup 1.0000x reference)
import jax
import jax.numpy as jnp
from jax.experimental import pallas as pl


def kernel(x_modality1, x_modality2, edge_index, W1, b1, W2, b2):
    raise NotImplementedError("write your pallas kernel here")


# speedup vs baseline: 23.6825x; 23.6825x over previous
import functools

import jax
import jax.numpy as jnp
from jax import lax
from jax.experimental import pallas as pl
from jax.experimental.pallas import tpu as pltpu
from jax.experimental.pallas import tpu_sc as plsc

_N = 10000
_D = 128
_NP = 10240
_C = 96
_NCH = 210
_EPAD = 16 * _NCH * _C
_ET16 = _EPAD // 16
_ET32 = _EPAD // 32
_ROWS_T = _NP // 16

_MESH = plsc.VectorSubcoreMesh(
    core_axis_name="c", subcore_axis_name="s", num_cores=2, num_subcores=16
)


def _deg_body(dst_hbm, deg_hbm, dst_v, deg_v):
    c = lax.axis_index("c")
    s = lax.axis_index("s")
    w = c * 16 + s
    pltpu.sync_copy(dst_hbm.at[pl.ds(w * _ET32, _ET32)], dst_v)

    zeros = jnp.zeros((16,), jnp.float32)

    def zero_body(i, carry):
        deg_v[pl.ds(i * 16, 16)] = zeros
        return carry

    lax.fori_loop(0, _NP // 16, zero_body, 0)

    ones = jnp.ones((16,), jnp.float32)

    def add_body(i, carry):
        idx = dst_v[pl.ds(i * 16, 16)]
        plsc.addupdate_scatter(deg_v, [idx], ones)
        return carry

    lax.fori_loop(0, _ET32 // 16, add_body, 0)
    pltpu.sync_copy(deg_v, deg_hbm.at[w])


_deg_call = pl.kernel(
    _deg_body,
    out_type=jax.ShapeDtypeStruct((32, _NP), jnp.float32),
    mesh=_MESH,
    compiler_params=pltpu.CompilerParams(needs_layout_passes=False),
    scratch_types=[
        pltpu.VMEM((_ET32,), jnp.int32),
        pltpu.VMEM((_NP,), jnp.float32),
    ],
)


def _mm_body(x1_ref, x2_ref, w1_ref, w2_ref, dp_ref, hs_ref):
    deg = jnp.sum(dp_ref[...], axis=0) + 1.0
    dinv = lax.rsqrt(deg)[:, None]
    h1 = jnp.dot(x1_ref[...], w1_ref[...], preferred_element_type=jnp.float32)
    h2 = jnp.dot(x2_ref[...], w2_ref[...], preferred_element_type=jnp.float32)
    hs_ref[0] = h1 * dinv
    hs_ref[1] = h2 * dinv


_RB = 1024

_mm_call = pl.pallas_call(
    _mm_body,
    grid=(_NP // _RB,),
    in_specs=[
        pl.BlockSpec((_RB, _D), lambda i: (i, 0)),
        pl.BlockSpec((_RB, _D), lambda i: (i, 0)),
        pl.BlockSpec((_D, _D), lambda i: (0, 0)),
        pl.BlockSpec((_D, _D), lambda i: (0, 0)),
        pl.BlockSpec((32, _RB), lambda i: (0, i)),
    ],
    out_specs=pl.BlockSpec((2, _RB, _D), lambda i: (0, i, 0)),
    out_shape=jax.ShapeDtypeStruct((2, _NP, _D), jnp.float32),
)


def _agg_body(hs_hbm, src_hbm, dst_hbm, out_hbm,
              src_v, dsti0, dsti1, rows0, rows1,
              gsem0, gsem1, dsem0, dsem1, acc):
    c = lax.axis_index("c")
    s = lax.axis_index("s")
    base = s * _ET16
    hs_c = hs_hbm.at[c]

    pltpu.sync_copy(src_hbm.at[pl.ds(base, _ET16)], src_v)
    r0 = s * _ROWS_T
    pltpu.sync_copy(hs_c.at[pl.ds(r0, _ROWS_T)], acc.at[pl.ds(r0, _ROWS_T)])
    plsc.subcore_barrier()

    def start(j, rows, dsti, gsem, dsem):
        pltpu.async_copy(hs_c.at[src_v.at[pl.ds(j * _C, _C)]], rows, gsem)
        pltpu.async_copy(dst_hbm.at[pl.ds(base + j * _C, _C)], dsti, dsem)

    def finish(j, rows, dsti, gsem, dsem):
        pltpu.make_async_copy(
            hs_c.at[src_v.at[pl.ds(j * _C, _C)]], rows, gsem).wait()
        pltpu.make_async_copy(
            dst_hbm.at[pl.ds(base + j * _C, _C)], dsti, dsem).wait()
        pltpu.sync_copy(rows, acc.at[dsti], add=True)

    start(0, rows0, dsti0, gsem0, dsem0)

    def body(i, carry):
        j0 = i * 2
        start(j0 + 1, rows1, dsti1, gsem1, dsem1)
        finish(j0, rows0, dsti0, gsem0, dsem0)

        @pl.when(j0 + 2 < _NCH)
        def _next():
            start(j0 + 2, rows0, dsti0, gsem0, dsem0)

        finish(j0 + 1, rows1, dsti1, gsem1, dsem1)
        return carry

    lax.fori_loop(0, _NCH // 2, body, 0)
    plsc.subcore_barrier()
    pltpu.sync_copy(acc.at[pl.ds(r0, _ROWS_T)],
                    out_hbm.at[c].at[pl.ds(r0, _ROWS_T)])


_agg_call = pl.kernel(
    _agg_body,
    out_type=jax.ShapeDtypeStruct((2, _NP, _D), jnp.float32),
    mesh=_MESH,
    compiler_params=pltpu.CompilerParams(needs_layout_passes=False),
    scratch_types=[
        pltpu.VMEM((_ET16,), jnp.int32),
        pltpu.VMEM((_C,), jnp.int32),
        pltpu.VMEM((_C,), jnp.int32),
        pltpu.VMEM((_C, _D), jnp.float32),
        pltpu.VMEM((_C, _D), jnp.float32),
        pltpu.SemaphoreType.DMA,
        pltpu.SemaphoreType.DMA,
        pltpu.SemaphoreType.DMA,
        pltpu.SemaphoreType.DMA,
        pltpu.VMEM_SHARED((_NP, _D), jnp.float32),
    ],
)


def _out_body(agg_ref, dp_ref, b1_ref, b2_ref, o_ref):
    deg = jnp.sum(dp_ref[...], axis=0) + 1.0
    dinv = lax.rsqrt(deg)[:, None]
    h1 = jnp.maximum(agg_ref[0] * dinv + b1_ref[...], 0.0)
    h2 = jnp.maximum(agg_ref[1] * dinv + b2_ref[...], 0.0)
    x = h1 + h2
    m = jnp.max(x, axis=1, keepdims=True)
    e = jnp.exp(x - m)
    o_ref[...] = x - (jnp.log(jnp.sum(e, axis=1, keepdims=True)) + m)


_out_call = pl.pallas_call(
    _out_body,
    grid=(_NP // _RB,),
    in_specs=[
        pl.BlockSpec((2, _RB, _D), lambda i: (0, i, 0)),
        pl.BlockSpec((32, _RB), lambda i: (0, i)),
        pl.BlockSpec((1, _D), lambda i: (0, 0)),
        pl.BlockSpec((1, _D), lambda i: (0, 0)),
    ],
    out_specs=pl.BlockSpec((_RB, _D), lambda i: (i, 0)),
    out_shape=jax.ShapeDtypeStruct((_NP, _D), jnp.float32),
)


def kernel(x_modality1, x_modality2, edge_index, W1, b1, W2, b2):
    n = x_modality1.shape[0]
    e = edge_index.shape[1]
    pad_e = _EPAD - e
    src_p = jnp.concatenate(
        [edge_index[0], jnp.full((pad_e,), _N, jnp.int32)])
    dst_p = jnp.concatenate(
        [edge_index[1], jnp.full((pad_e,), _N, jnp.int32)])
    x1p = jnp.pad(x_modality1, ((0, _NP - n), (0, 0)))
    x2p = jnp.pad(x_modality2, ((0, _NP - n), (0, 0)))

    deg_parts = _deg_call(dst_p)
    hs = _mm_call(x1p, x2p, W1, W2, deg_parts)
    agg = _agg_call(hs, src_p, dst_p)
    out = _out_call(agg, deg_parts,
                    b1.reshape(1, _D), b2.reshape(1, _D))
    return out[:n]
